# baseline (speedup 1.0000x reference)
import functools

import jax
import jax.numpy as jnp
from jax import lax
from jax.experimental import pallas as pl
from jax.experimental.pallas import tpu as pltpu
from jax.experimental.pallas import tpu_sc as plsc

N = 10000
E = 320000
D_IN = 128
D_H = 128
D_OUT = 64

N_PAD = 10240
BN = 2048
GRID = N_PAD // BN
NW = 32
CH = 128
NCH = 80
EPW = NCH * CH
E_PAD = NW * EPW
RPS = N_PAD // 16


def _seg_body(d, with_deg, nbuf, ih, *refs):
    nchh = NCH // ih
    ngh = nchh // nbuf
    if with_deg:
        (p_hbm, idx_hbm, acc_out, deg_out,
         idx_v, rows_a, ones_v, dz_v, acc_sh, deg_sh,
         sem_i, *sems) = refs
        sem_g = sems[0:nbuf]
        sem_s = sems[nbuf:2 * nbuf]
        sem_o = sems[2 * nbuf:3 * nbuf]
    else:
        (p_hbm, idx_hbm, acc_out,
         idx_v, rows_a, acc_sh,
         sem_i, *sems) = refs
        sem_g = sems[0:nbuf]
        sem_s = sems[nbuf:2 * nbuf]
    core = lax.axis_index("c")
    sid = lax.axis_index("s")
    wid = sid * 2 + core

    idx_load = pltpu.async_copy(idx_hbm.at[pl.ds(wid * NCH, nchh)], idx_v,
                                sem_i)

    z16 = jnp.zeros((16,), jnp.float32)
    o16 = jnp.ones((16,), jnp.float32)

    def zrow(i, _):
        rows_a[0, i // (d // 16), pl.ds((i % (d // 16)) * 16, 16)] = z16
        return _
    lax.fori_loop(0, CH * (d // 16), zrow, None)

    rbase = sid * RPS

    def zcp(t, _):
        pltpu.sync_copy(rows_a.at[0], acc_sh.at[pl.ds(rbase + t * CH, CH)])
        return _
    lax.fori_loop(0, RPS // CH, zcp, None)

    if with_deg:
        def zdeg(i, _):
            dz_v[pl.ds(i * 16, 16)] = z16
            return _
        lax.fori_loop(0, RPS // 16, zdeg, None)

        def fill1(i, _):
            ones_v[pl.ds(i * 16, 16)] = o16
            return _
        lax.fori_loop(0, CH // 16, fill1, None)
        pltpu.sync_copy(dz_v, deg_sh.at[pl.ds(rbase, RPS)])

    idx_load.wait()
    plsc.subcore_barrier()

    def gather(j, k):
        return pltpu.async_copy(p_hbm.at[idx_v.at[j, 0]], rows_a.at[k],
                                sem_g[k])

    def wait_gather(j, k):
        pltpu.make_async_copy(p_hbm.at[idx_v.at[j, 0]], rows_a.at[k],
                              sem_g[k]).wait()

    def scat(j, k):
        pltpu.async_copy(rows_a.at[k], acc_sh.at[idx_v.at[j, 1]], sem_s[k],
                         add=True)
        if with_deg:
            pltpu.async_copy(ones_v, deg_sh.at[idx_v.at[j, 1]], sem_o[k],
                             add=True)

    def wait_scat(j, k):
        pltpu.make_async_copy(rows_a.at[k], acc_sh.at[idx_v.at[j, 1]],
                              sem_s[k]).wait()
        if with_deg:
            pltpu.make_async_copy(ones_v, deg_sh.at[idx_v.at[j, 1]],
                                  sem_o[k]).wait()

    def run_half():
        for k in range(nbuf):
            gather(k, k)

        def group(j2, _):
            base = j2 * nbuf
            for k in range(nbuf):
                wait_gather(base + k, k)
                scat(base + k, k)
            for k in range(nbuf):
                wait_scat(base + k, k)
                gather(base + nbuf + k, k)
            return _
        lax.fori_loop(0, ngh - 1, group, None)

        last = (ngh - 1) * nbuf
        for k in range(nbuf):
            wait_gather(last + k, k)
            scat(last + k, k)
        for k in range(nbuf):
            wait_scat(last + k, k)

    run_half()
    for h in range(1, ih):
        pltpu.sync_copy(idx_hbm.at[pl.ds(wid * NCH + h * nchh, nchh)], idx_v)
        run_half()

    plsc.subcore_barrier()

    pltpu.sync_copy(acc_sh.at[pl.ds(rbase, RPS)],
                    acc_out.at[core, pl.ds(rbase, RPS)])
    if with_deg:
        pltpu.sync_copy(deg_sh.at[pl.ds(rbase, RPS)],
                        deg_out.at[core, pl.ds(rbase, RPS)])


def _make_segsum(d, with_deg, nbuf, ih):
    mesh = plsc.VectorSubcoreMesh(core_axis_name="c", subcore_axis_name="s")
    out_type = [jax.ShapeDtypeStruct((2, N_PAD, d), jnp.float32)]
    scratch = [
        pltpu.VMEM((NCH // ih, 2, CH), jnp.int32),
        pltpu.VMEM((nbuf, CH, d), jnp.float32),
    ]
    if with_deg:
        out_type.append(jax.ShapeDtypeStruct((2, N_PAD), jnp.float32))
        scratch += [
            pltpu.VMEM((CH,), jnp.float32),
            pltpu.VMEM((RPS,), jnp.float32),
        ]
    scratch.append(pltpu.VMEM_SHARED((N_PAD, d), jnp.float32))
    if with_deg:
        scratch.append(pltpu.VMEM_SHARED((N_PAD,), jnp.float32))
    n_sems = 1 + (3 if with_deg else 2) * nbuf
    scratch += [pltpu.SemaphoreType.DMA] * n_sems
    return pl.kernel(
        functools.partial(_seg_body, d, with_deg, nbuf, ih),
        out_type=tuple(out_type),
        mesh=mesh,
        scratch_types=scratch,
        compiler_params=pltpu.CompilerParams(use_tc_tiling_on_sc=False),
        name=f"sage_segsum_d{d}",
    )


_segsum128 = _make_segsum(D_H, True, 2, 2)
_segsum64 = _make_segsum(D_OUT, False, 4, 1)


def _tc1_body(x_ref, wn_ref, ws_ref, b_ref, p1_ref, xws_ref):
    xb = x_ref[...]
    p1_ref[...] = jnp.dot(xb, wn_ref[...], preferred_element_type=jnp.float32)
    xws_ref[...] = (jnp.dot(xb, ws_ref[...], preferred_element_type=jnp.float32)
                    + b_ref[...])


_tc1 = pl.pallas_call(
    _tc1_body,
    grid=(GRID,),
    in_specs=[
        pl.BlockSpec((BN, D_IN), lambda i: (i, 0)),
        pl.BlockSpec((D_IN, D_H), lambda i: (0, 0)),
        pl.BlockSpec((D_IN, D_H), lambda i: (0, 0)),
        pl.BlockSpec((1, D_H), lambda i: (0, 0)),
    ],
    out_specs=[
        pl.BlockSpec((BN, D_H), lambda i: (i, 0)),
        pl.BlockSpec((BN, D_H), lambda i: (i, 0)),
    ],
    out_shape=[
        jax.ShapeDtypeStruct((N_PAD, D_H), jnp.float32),
        jax.ShapeDtypeStruct((N_PAD, D_H), jnp.float32),
    ],
    name="sage_tc1",
)


def _tc2_body(xws_ref, acc_ref, deg_ref, ws2_ref, wn2_ref, b2_ref,
              p2_ref, hws_ref):
    a = acc_ref[0, :, :] + acc_ref[1, :, :]
    dsum = deg_ref[0, :, :] + deg_ref[1, :, :]
    inv = 1.0 / jnp.maximum(dsum, 1.0)
    h = jnp.maximum(xws_ref[...] + a * inv, 0.0)
    p2_ref[...] = jnp.dot(h, wn2_ref[...], preferred_element_type=jnp.float32)
    hws_ref[...] = (jnp.dot(h, ws2_ref[...], preferred_element_type=jnp.float32)
                    + b2_ref[...])


_tc2 = pl.pallas_call(
    _tc2_body,
    grid=(GRID,),
    in_specs=[
        pl.BlockSpec((BN, D_H), lambda i: (i, 0)),
        pl.BlockSpec((2, BN, D_H), lambda i: (0, i, 0)),
        pl.BlockSpec((2, BN, 1), lambda i: (0, i, 0)),
        pl.BlockSpec((D_H, D_OUT), lambda i: (0, 0)),
        pl.BlockSpec((D_H, D_OUT), lambda i: (0, 0)),
        pl.BlockSpec((1, D_OUT), lambda i: (0, 0)),
    ],
    out_specs=[
        pl.BlockSpec((BN, D_OUT), lambda i: (i, 0)),
        pl.BlockSpec((BN, D_OUT), lambda i: (i, 0)),
    ],
    out_shape=[
        jax.ShapeDtypeStruct((N_PAD, D_OUT), jnp.float32),
        jax.ShapeDtypeStruct((N_PAD, D_OUT), jnp.float32),
    ],
    name="sage_tc2",
)


def _tc3_body(hws_ref, acc_ref, deg_ref, out_ref):
    a = acc_ref[0, :, :] + acc_ref[1, :, :]
    dsum = deg_ref[0, :, :] + deg_ref[1, :, :]
    inv = 1.0 / jnp.maximum(dsum, 1.0)
    out_ref[...] = hws_ref[...] + a * inv


_tc3 = pl.pallas_call(
    _tc3_body,
    grid=(GRID,),
    in_specs=[
        pl.BlockSpec((BN, D_OUT), lambda i: (i, 0)),
        pl.BlockSpec((2, BN, D_OUT), lambda i: (0, i, 0)),
        pl.BlockSpec((2, BN, 1), lambda i: (0, i, 0)),
    ],
    out_specs=pl.BlockSpec((BN, D_OUT), lambda i: (i, 0)),
    out_shape=jax.ShapeDtypeStruct((N_PAD, D_OUT), jnp.float32),
    name="sage_tc3",
)


def kernel(x, edge_index, W_self1, W_neigh1, b1, W_self2, W_neigh2, b2):
    src = edge_index[0]
    dst = edge_index[1]
    pad = E_PAD - E
    src_p = jnp.concatenate([src, jnp.zeros((pad,), jnp.int32)])
    dst_p = jnp.concatenate([dst, jnp.full((pad,), N, jnp.int32)])
    idx_pack = jnp.stack(
        [src_p.reshape(-1, CH), dst_p.reshape(-1, CH)], axis=1)
    x_p = jnp.pad(x, ((0, N_PAD - N), (0, 0)))

    p1, xws1 = _tc1(x_p, W_neigh1, W_self1, b1.reshape(1, D_H))
    acc1, deg = _segsum128(p1, idx_pack)
    deg3 = deg.reshape(2, N_PAD, 1)
    p2, hws2 = _tc2(xws1, acc1, deg3, W_self2, W_neigh2, b2.reshape(1, D_OUT))
    (acc2,) = _segsum64(p2, idx_pack)
    out = _tc3(hws2, acc2, deg3)
    return out[:N]
